# Initial kernel scaffold; baseline (speedup 1.0000x reference)
import jax
import jax.numpy as jnp
from jax.experimental import pallas as pl


def kernel(x, edge_index):
    raise NotImplementedError("write your pallas kernel here")


# speedup vs baseline: 19.9056x; 19.9056x over previous
import functools

import jax
import jax.numpy as jnp
from jax import lax
from jax.experimental import pallas as pl
from jax.experimental.pallas import tpu as pltpu
from jax.experimental.pallas import tpu_sc as plsc

D = 128
L = 16
NC = 2
NS = 16
NW = NC * NS
K = 128


def _sc_segment_sum(x, row3, col3, n_pad, nch):
    rpt = n_pad // NS

    mesh = plsc.VectorSubcoreMesh(core_axis_name="c", subcore_axis_name="s")

    @functools.partial(
        pl.kernel,
        out_type=jax.ShapeDtypeStruct((NC, n_pad, D), jnp.float32),
        mesh=mesh,
        scratch_types=[
            pltpu.VMEM((nch, K), jnp.int32),
            pltpu.VMEM((nch, K), jnp.int32),
            pltpu.VMEM((K, D), jnp.float32),
            pltpu.VMEM_SHARED((n_pad, D), jnp.float32),
            pltpu.SemaphoreType.DMA,
        ],
    )
    def seg_sum(x_hbm, row_hbm, col_hbm, out_hbm, row_v, col_v, rows_v, acc_sh, sem):
        cid = lax.axis_index("c")
        sid = lax.axis_index("s")
        tid = cid * NS + sid

        zeros16 = jnp.zeros((L,), jnp.float32)

        def zero_row(r):
            for c in range(0, D, L):
                rows_v[r, pl.ds(c, L)] = zeros16

        pl.loop(0, K)(zero_row)

        base = sid * rpt
        off = 0
        while off < rpt:
            n = min(K, rpt - off)
            pltpu.sync_copy(rows_v.at[pl.ds(0, n)], acc_sh.at[pl.ds(base + off, n)])
            off += n

        plsc.subcore_barrier()

        pltpu.sync_copy(row_hbm.at[tid], row_v)
        pltpu.sync_copy(col_hbm.at[tid], col_v)

        def chunk(j):
            pltpu.async_copy(x_hbm.at[row_v.at[j]], rows_v, sem).wait()
            pltpu.sync_copy(rows_v, acc_sh.at[col_v.at[j]], add=True)

        pl.loop(0, nch)(chunk)

        plsc.subcore_barrier()

        pltpu.sync_copy(acc_sh.at[pl.ds(base, rpt)],
                        out_hbm.at[cid, pl.ds(base, rpt)])

    return seg_sum(x, row3, col3)


def _tc_normalize(partials, n):
    br = 1000
    assert n % br == 0

    def body(p_ref, o_ref):
        p = p_ref[0] + p_ref[1]
        lns = 2.0 * p[:, 0] * p[:, 0] - jnp.sum(p * p, axis=1)
        bad = lns <= 1e-8
        basepoint = (lax.broadcasted_iota(jnp.int32, (br, D), 1) == 0)
        p = jnp.where(bad[:, None], basepoint.astype(jnp.float32), p)
        lns = jnp.where(bad, 1.0, lns)
        denom = jnp.maximum(jnp.sqrt(jnp.maximum(lns, 0.0)), 1e-12)
        out = p / denom[:, None]
        o_ref[...] = jnp.where((out[:, 0] <= 0.0)[:, None], -out, out)

    return pl.pallas_call(
        body,
        grid=(n // br,),
        in_specs=[pl.BlockSpec((2, br, D), lambda i: (0, i, 0))],
        out_specs=pl.BlockSpec((br, D), lambda i: (i, 0)),
        out_shape=jax.ShapeDtypeStruct((n, D), jnp.float32),
    )(partials)


def kernel(x, edge_index):
    n = x.shape[0]
    e = edge_index.shape[1]

    nch = -(-e // (NW * K))
    e_pad = NW * K * nch
    n_pad = -(-(n + 1) // (NS * 8)) * (NS * 8)

    row = edge_index[0]
    col = edge_index[1]
    pad = e_pad - e
    row3 = jnp.concatenate([row, jnp.zeros((pad,), jnp.int32)]).reshape(NW, nch, K)
    col3 = jnp.concatenate([col, jnp.full((pad,), n, jnp.int32)]).reshape(NW, nch, K)

    partials = _sc_segment_sum(x, row3, col3, n_pad, nch)
    return _tc_normalize(partials[:, :n, :], n)


# speedup vs baseline: 23.4133x; 1.1762x over previous
import functools

import jax
import jax.numpy as jnp
from jax import lax
from jax.experimental import pallas as pl
from jax.experimental.pallas import tpu as pltpu
from jax.experimental.pallas import tpu_sc as plsc

D = 128
L = 16
NC = 2
NS = 16
NW = NC * NS
K = 128


def _sc_segment_sum(x, row3, col3, n_pad, nch):
    rpt = n_pad // NS

    mesh = plsc.VectorSubcoreMesh(core_axis_name="c", subcore_axis_name="s")

    @functools.partial(
        pl.kernel,
        out_type=jax.ShapeDtypeStruct((NC, n_pad, D), jnp.float32),
        mesh=mesh,
        scratch_types=[
            pltpu.VMEM((K,), jnp.int32),
            pltpu.VMEM((K,), jnp.int32),
            pltpu.VMEM((K,), jnp.int32),
            pltpu.VMEM((K,), jnp.int32),
            pltpu.VMEM((K, D), jnp.float32),
            pltpu.VMEM((K, D), jnp.float32),
            pltpu.VMEM_SHARED((n_pad, D), jnp.float32),
            pltpu.SemaphoreType.DMA,
            pltpu.SemaphoreType.DMA,
            pltpu.SemaphoreType.DMA,
            pltpu.SemaphoreType.DMA,
        ],
    )
    def seg_sum(x_hbm, row_hbm, col_hbm, out_hbm,
                ri0, ri1, ci0, ci1, b0, b1, acc_sh, gs0, gs1, is0, is1):
        cid = lax.axis_index("c")
        sid = lax.axis_index("s")
        tid = cid * NS + sid

        zeros16 = jnp.zeros((L,), jnp.float32)

        def zero_row(r):
            for c in range(0, D, L):
                b0[r, pl.ds(c, L)] = zeros16

        pl.loop(0, K)(zero_row)

        base = sid * rpt
        off = 0
        while off < rpt:
            n = min(K, rpt - off)
            pltpu.sync_copy(b0.at[pl.ds(0, n)], acc_sh.at[pl.ds(base + off, n)])
            off += n

        plsc.subcore_barrier()

        pltpu.sync_copy(row_hbm.at[tid, 0], ri0)
        pltpu.sync_copy(col_hbm.at[tid, 0], ci0)
        pltpu.async_copy(x_hbm.at[ri0], b0, gs0)
        pltpu.async_copy(row_hbm.at[tid, 1], ri1, is1)
        pltpu.async_copy(col_hbm.at[tid, 1], ci1, is1)

        def step(j, rb_p, cb_p, buf_p, gs_p, is_p, rb_n, cb_n, buf_n, gs_n, is_n):
            @pl.when(j + 1 < nch)
            def _():
                pltpu.make_async_copy(row_hbm.at[tid, 0], rb_n, is_n).wait()
                pltpu.make_async_copy(col_hbm.at[tid, 0], cb_n, is_n).wait()
                pltpu.async_copy(x_hbm.at[rb_n], buf_n, gs_n)

            pltpu.make_async_copy(x_hbm.at[rb_p], buf_p, gs_p).wait()
            pltpu.sync_copy(buf_p, acc_sh.at[cb_p], add=True)

            @pl.when(j + 2 < nch)
            def _():
                pltpu.async_copy(row_hbm.at[tid, j + 2], rb_p, is_p)
                pltpu.async_copy(col_hbm.at[tid, j + 2], cb_p, is_p)

        def chunk(j):
            @pl.when(j % 2 == 0)
            def _():
                step(j, ri0, ci0, b0, gs0, is0, ri1, ci1, b1, gs1, is1)

            @pl.when(j % 2 == 1)
            def _():
                step(j, ri1, ci1, b1, gs1, is1, ri0, ci0, b0, gs0, is0)

        pl.loop(0, nch)(chunk)

        plsc.subcore_barrier()

        pltpu.sync_copy(acc_sh.at[pl.ds(base, rpt)],
                        out_hbm.at[cid, pl.ds(base, rpt)])

    return seg_sum(x, row3, col3)


def _tc_normalize(partials, n):
    br = 1000
    assert n % br == 0

    def body(p_ref, o_ref):
        p = p_ref[0] + p_ref[1]
        lns = 2.0 * p[:, 0] * p[:, 0] - jnp.sum(p * p, axis=1)
        bad = lns <= 1e-8
        basepoint = (lax.broadcasted_iota(jnp.int32, (br, D), 1) == 0)
        p = jnp.where(bad[:, None], basepoint.astype(jnp.float32), p)
        lns = jnp.where(bad, 1.0, lns)
        denom = jnp.maximum(jnp.sqrt(jnp.maximum(lns, 0.0)), 1e-12)
        out = p / denom[:, None]
        o_ref[...] = jnp.where((out[:, 0] <= 0.0)[:, None], -out, out)

    return pl.pallas_call(
        body,
        grid=(n // br,),
        in_specs=[pl.BlockSpec((2, br, D), lambda i: (0, i, 0))],
        out_specs=pl.BlockSpec((br, D), lambda i: (i, 0)),
        out_shape=jax.ShapeDtypeStruct((n, D), jnp.float32),
    )(partials)


def kernel(x, edge_index):
    n = x.shape[0]
    e = edge_index.shape[1]

    nch = -(-e // (NW * K))
    e_pad = NW * K * nch
    n_pad = -(-(n + 1) // (NS * 8)) * (NS * 8)

    row = edge_index[0]
    col = edge_index[1]
    pad = e_pad - e
    row3 = jnp.concatenate([row, jnp.zeros((pad,), jnp.int32)]).reshape(NW, nch, K)
    col3 = jnp.concatenate([col, jnp.full((pad,), n, jnp.int32)]).reshape(NW, nch, K)

    partials = _sc_segment_sum(x, row3, col3, n_pad, nch)
    return _tc_normalize(partials[:, :n, :], n)
